# baseline (speedup 1.0000x reference)
import functools

import jax
import jax.numpy as jnp
from jax import lax
from jax.experimental import pallas as pl
from jax.experimental.pallas import tpu as pltpu
from jax.experimental.pallas import tpu_sc as plsc

BATCH = 4096
STATE_DIM = 512
HIDDEN = 1024
FEATURE_DIM = 128
N_ABS = 100000


_BM = 1024


def _mlp_body(x_ref, w1_ref, b1_ref, w2_ref, b2_ref, o_ref):
    h = jnp.dot(x_ref[...], w1_ref[...], preferred_element_type=jnp.float32)
    h = jnp.maximum(h + b1_ref[...], 0.0)
    o_ref[...] = (
        jnp.dot(h, w2_ref[...], preferred_element_type=jnp.float32) + b2_ref[...]
    )


def _mlp_part(states, W1, b1, W2, b2, row0, nrows):
    grid = (nrows // _BM,)
    blk0 = row0 // _BM
    return pl.pallas_call(
        _mlp_body,
        grid=grid,
        in_specs=[
            pl.BlockSpec((_BM, STATE_DIM), lambda i: (i + blk0, 0)),
            pl.BlockSpec((STATE_DIM, HIDDEN), lambda i: (0, 0)),
            pl.BlockSpec((1, HIDDEN), lambda i: (0, 0)),
            pl.BlockSpec((HIDDEN, FEATURE_DIM), lambda i: (0, 0)),
            pl.BlockSpec((1, FEATURE_DIM), lambda i: (0, 0)),
        ],
        out_specs=pl.BlockSpec((_BM, FEATURE_DIM), lambda i: (i, 0)),
        out_shape=jax.ShapeDtypeStruct((nrows, FEATURE_DIM), jnp.float32),
    )(states, W1, b1.reshape(1, HIDDEN), W2, b2.reshape(1, FEATURE_DIM))


_NC = 2
_NS = 16
_NW = _NC * _NS
_B_PER_W = BATCH // _NW


@functools.partial(
    pl.kernel,
    mesh=plsc.VectorSubcoreMesh(core_axis_name="c", subcore_axis_name="s"),
    out_type=jax.ShapeDtypeStruct((BATCH, FEATURE_DIM), jnp.float32),
    scratch_types=[
        pltpu.VMEM((_B_PER_W,), jnp.int32),
        pltpu.VMEM((_B_PER_W, FEATURE_DIM), jnp.float32),
        pltpu.SemaphoreType.DMA,
    ],
)
def _gather_sc(table_hbm, idx_hbm, out_hbm, idx_v, rows_v, sem):
    wid = lax.axis_index("s") * _NC + lax.axis_index("c")
    base = wid * _B_PER_W
    pltpu.sync_copy(idx_hbm.at[pl.ds(base, _B_PER_W)], idx_v)
    pltpu.async_copy(table_hbm.at[idx_v], rows_v, sem).wait()
    pltpu.sync_copy(rows_v, out_hbm.at[pl.ds(base, _B_PER_W)])


def kernel(states, indices, W1, b1, W2, b2, embed_table):
    half = BATCH // 2
    xs_a = _mlp_part(states, W1, b1, W2, b2, 0, half)
    idx = lax.optimization_barrier((indices, xs_a))[0]
    embeds = _gather_sc(embed_table, idx)
    xs_b = _mlp_part(states, W1, b1, W2, b2, half, BATCH - half)
    xs = jnp.concatenate([xs_a, xs_b], axis=0)
    return (xs, embeds)


# speedup vs baseline: 1.1836x; 1.1836x over previous
import functools

import jax
import jax.numpy as jnp
from jax import lax
from jax.experimental import pallas as pl
from jax.experimental.pallas import tpu as pltpu
from jax.experimental.pallas import tpu_sc as plsc

BATCH = 4096
STATE_DIM = 512
HIDDEN = 1024
FEATURE_DIM = 128
N_ABS = 100000


_BM = 1024


def _mlp_body(x_ref, w1_ref, b1_ref, w2_ref, b2_ref, o_ref):
    h = jnp.dot(x_ref[...], w1_ref[...], preferred_element_type=jnp.float32)
    h = jnp.maximum(h + b1_ref[...], 0.0)
    o_ref[...] = (
        jnp.dot(h, w2_ref[...], preferred_element_type=jnp.float32) + b2_ref[...]
    )


def _mlp_part(states, W1, b1, W2, b2, row0, nrows):
    grid = (nrows // _BM,)
    blk0 = row0 // _BM
    return pl.pallas_call(
        _mlp_body,
        grid=grid,
        in_specs=[
            pl.BlockSpec((_BM, STATE_DIM), lambda i: (i + blk0, 0)),
            pl.BlockSpec((STATE_DIM, HIDDEN), lambda i: (0, 0)),
            pl.BlockSpec((1, HIDDEN), lambda i: (0, 0)),
            pl.BlockSpec((HIDDEN, FEATURE_DIM), lambda i: (0, 0)),
            pl.BlockSpec((1, FEATURE_DIM), lambda i: (0, 0)),
        ],
        out_specs=pl.BlockSpec((_BM, FEATURE_DIM), lambda i: (i, 0)),
        out_shape=jax.ShapeDtypeStruct((nrows, FEATURE_DIM), jnp.float32),
    )(states, W1, b1.reshape(1, HIDDEN), W2, b2.reshape(1, FEATURE_DIM))


_NC = 2
_NS = 16
_NW = _NC * _NS
_B_PER_W = BATCH // _NW


@functools.partial(
    pl.kernel,
    mesh=plsc.VectorSubcoreMesh(core_axis_name="c", subcore_axis_name="s"),
    out_type=jax.ShapeDtypeStruct((BATCH, FEATURE_DIM), jnp.float32),
    scratch_types=[
        pltpu.VMEM((_B_PER_W,), jnp.int32),
        pltpu.VMEM((_B_PER_W, FEATURE_DIM), jnp.float32),
        pltpu.SemaphoreType.DMA,
    ],
)
def _gather_sc(table_hbm, idx_hbm, out_hbm, idx_v, rows_v, sem):
    wid = lax.axis_index("s") * _NC + lax.axis_index("c")
    base = wid * _B_PER_W
    pltpu.sync_copy(idx_hbm.at[pl.ds(base, _B_PER_W)], idx_v)
    pltpu.async_copy(table_hbm.at[idx_v], rows_v, sem).wait()
    pltpu.sync_copy(rows_v, out_hbm.at[pl.ds(base, _B_PER_W)])


def kernel(states, indices, W1, b1, W2, b2, embed_table):
    embeds = _gather_sc(embed_table, indices)
    xs = _mlp_part(states, W1, b1, W2, b2, 0, BATCH)
    return (xs, embeds)
